# baseline (speedup 1.0000x reference)
import functools

import jax
import jax.numpy as jnp
from jax import lax
from jax.experimental import pallas as pl
from jax.experimental.pallas import tpu as pltpu
from jax.experimental.pallas import tpu_sc as plsc

_B, _T, _D = 4, 8192, 1024
_NFLAT = _B * _T * _D
_PFLAT = _T * _D
_NW = 32
_PER_W = _NFLAT // _NW
_CH = 16384
_NCHUNK = _PER_W // _CH
_LANES = 16


def _sc_body(x_hbm, pos_hbm, out_hbm, xb0, pb0, xb1, pb1, sem_in, sem_out):
    c = lax.axis_index("c")
    s = lax.axis_index("s")
    wid = s * 2 + c
    base = wid * _PER_W
    pos_base = lax.rem(wid, 8) * _PER_W

    def fetch(i, xb, pb):
        off = base + i * _CH
        poff = pos_base + i * _CH
        cp_x = pltpu.make_async_copy(x_hbm.at[pl.ds(off, _CH)], xb, sem_in)
        cp_p = pltpu.make_async_copy(pos_hbm.at[pl.ds(poff, _CH)], pb, sem_in)
        cp_x.start()
        cp_p.start()
        return cp_x, cp_p

    def compute(xb, pb):
        def vbody(j, _):
            sl = pl.ds(j * _LANES, _LANES)
            xb[sl] = xb[sl] + pb[sl]
            return 0
        lax.fori_loop(0, _CH // _LANES, vbody, 0, unroll=8)

    def wait_in(xb, pb):
        pltpu.make_async_copy(x_hbm.at[pl.ds(base, _CH)], xb, sem_in).wait()
        pltpu.make_async_copy(pos_hbm.at[pl.ds(base, _CH)], pb, sem_in).wait()

    def wait_out(xb):
        pltpu.make_async_copy(xb, out_hbm.at[pl.ds(base, _CH)], sem_out).wait()

    def step(i, xb, pb, xo, po):
        wait_in(xb, pb)
        @pl.when(i >= 1)
        def _():
            wait_out(xo)

        @pl.when(i + 1 < _NCHUNK)
        def _():
            fetch(i + 1, xo, po)
        compute(xb, pb)
        pltpu.make_async_copy(xb, out_hbm.at[pl.ds(base + i * _CH, _CH)], sem_out).start()

    fetch(0, xb0, pb0)

    def loop(k, _):
        i = k * 2
        step(i, xb0, pb0, xb1, pb1)
        step(i + 1, xb1, pb1, xb0, pb0)
        return 0

    lax.fori_loop(0, _NCHUNK // 2, loop, 0)
    wait_out(xb1)


def kernel(x, pos_table):
    B, T, D = x.shape
    xf = x.reshape(-1)
    pf = pos_table[:T].reshape(-1)
    mesh = plsc.VectorSubcoreMesh(core_axis_name="c", subcore_axis_name="s")
    run = pl.kernel(
        _sc_body,
        out_type=jax.ShapeDtypeStruct((_NFLAT,), jnp.float32),
        mesh=mesh,
        scratch_types=[
            pltpu.VMEM((_CH,), jnp.float32),
            pltpu.VMEM((_CH,), jnp.float32),
            pltpu.VMEM((_CH,), jnp.float32),
            pltpu.VMEM((_CH,), jnp.float32),
            pltpu.SemaphoreType.DMA,
            pltpu.SemaphoreType.DMA,
        ],
    )
    out = run(xf, pf)
    return out.reshape(B, T, D)


# speedup vs baseline: 1.4471x; 1.4471x over previous
import functools

import jax
import jax.numpy as jnp
from jax import lax
from jax.experimental import pallas as pl
from jax.experimental.pallas import tpu as pltpu
from jax.experimental.pallas import tpu_sc as plsc

_B, _T, _D = 4, 8192, 1024
_NFLAT = _B * _T * _D
_PFLAT = _T * _D
_NW = 32
_PER_W = _NFLAT // _NW
_CH = 16384
_NCHUNK = _PER_W // _CH
_LANES = 16


def _sc_body(x_hbm, pos_hbm, out_hbm, xb0, pb0, xb1, pb1, sem_in, sem_out):
    c = lax.axis_index("c")
    s = lax.axis_index("s")
    wid = s * 2 + c
    base = wid * _PER_W
    pos_base = lax.rem(wid, 8) * _PER_W

    def fetch(i, xb, pb):
        off = base + i * _CH
        poff = pos_base + i * _CH
        cp_x = pltpu.make_async_copy(x_hbm.at[pl.ds(off, _CH)], xb, sem_in)
        cp_p = pltpu.make_async_copy(pos_hbm.at[pl.ds(poff, _CH)], pb, sem_in)
        cp_x.start()
        cp_p.start()
        return cp_x, cp_p

    def compute(xb, pb):
        @plsc.parallel_loop(0, _CH // _LANES, step=1, unroll=8)
        def _vbody(j):
            sl = pl.ds(j * _LANES, _LANES)
            xb[sl] = xb[sl] + pb[sl]

    def wait_in(xb, pb):
        pltpu.make_async_copy(x_hbm.at[pl.ds(base, _CH)], xb, sem_in).wait()
        pltpu.make_async_copy(pos_hbm.at[pl.ds(base, _CH)], pb, sem_in).wait()

    def wait_out(xb):
        pltpu.make_async_copy(xb, out_hbm.at[pl.ds(base, _CH)], sem_out).wait()

    def step(i, xb, pb, xo, po):
        wait_in(xb, pb)
        @pl.when(i >= 1)
        def _():
            wait_out(xo)

        @pl.when(i + 1 < _NCHUNK)
        def _():
            fetch(i + 1, xo, po)
        compute(xb, pb)
        pltpu.make_async_copy(xb, out_hbm.at[pl.ds(base + i * _CH, _CH)], sem_out).start()

    fetch(0, xb0, pb0)

    def loop(k, _):
        i = k * 2
        step(i, xb0, pb0, xb1, pb1)
        step(i + 1, xb1, pb1, xb0, pb0)
        return 0

    lax.fori_loop(0, _NCHUNK // 2, loop, 0)
    wait_out(xb1)


def kernel(x, pos_table):
    B, T, D = x.shape
    xf = x.reshape(-1)
    pf = pos_table[:T].reshape(-1)
    mesh = plsc.VectorSubcoreMesh(core_axis_name="c", subcore_axis_name="s")
    run = pl.kernel(
        _sc_body,
        out_type=jax.ShapeDtypeStruct((_NFLAT,), jnp.float32),
        mesh=mesh,
        scratch_types=[
            pltpu.VMEM((_CH,), jnp.float32),
            pltpu.VMEM((_CH,), jnp.float32),
            pltpu.VMEM((_CH,), jnp.float32),
            pltpu.VMEM((_CH,), jnp.float32),
            pltpu.SemaphoreType.DMA,
            pltpu.SemaphoreType.DMA,
        ],
    )
    out = run(xf, pf)
    return out.reshape(B, T, D)


# speedup vs baseline: 3.6492x; 2.5217x over previous
import jax
import jax.numpy as jnp
from jax import lax
from jax.experimental import pallas as pl
from jax.experimental.pallas import tpu as pltpu
from jax.experimental.pallas import tpu_sc as plsc

_B, _T, _D = 4, 8192, 1024
_ROWS = _B * _T
_NW = 32
_PER_W = _ROWS // _NW
_CHR = 16
_NCHUNK = _PER_W // _CHR
_LANES = 16
_VPR = _D // _LANES


def _sc_body(x_hbm, pos_hbm, out_hbm, xb0, pb0, xb1, pb1, sem_in, sem_out):
    c = lax.axis_index("c")
    s = lax.axis_index("s")
    wid = s * 2 + c
    base = wid * _PER_W
    pos_base = lax.rem(wid, 8) * _PER_W

    def fetch(i, xb, pb):
        ro = base + i * _CHR
        po = pos_base + i * _CHR
        pltpu.make_async_copy(x_hbm.at[pl.ds(ro, _CHR), :], xb, sem_in).start()
        pltpu.make_async_copy(pos_hbm.at[pl.ds(po, _CHR), :], pb, sem_in).start()

    def wait_in(xb, pb):
        pltpu.make_async_copy(x_hbm.at[pl.ds(base, _CHR), :], xb, sem_in).wait()
        pltpu.make_async_copy(pos_hbm.at[pl.ds(base, _CHR), :], pb, sem_in).wait()

    def wait_out(xb):
        pltpu.make_async_copy(xb, out_hbm.at[pl.ds(base, _CHR), :], sem_out).wait()

    def compute(xb, pb):
        @plsc.parallel_loop(0, _CHR * _VPR, step=1, unroll=8)
        def _vbody(j):
            r = lax.shift_right_logical(j, 6)
            col = pl.multiple_of(
                lax.shift_left(lax.bitwise_and(j, _VPR - 1), 4), _LANES
            )
            sl = pl.ds(col, _LANES)
            xb[r, sl] = xb[r, sl] + pb[r, sl]

    def step(i, xb, pb, xo, po):
        wait_in(xb, pb)
        @pl.when(i >= 1)
        def _():
            wait_out(xo)

        @pl.when(i + 1 < _NCHUNK)
        def _():
            fetch(i + 1, xo, po)
        compute(xb, pb)
        pltpu.make_async_copy(
            xb, out_hbm.at[pl.ds(base + i * _CHR, _CHR), :], sem_out
        ).start()

    fetch(0, xb0, pb0)

    def loop(k, _):
        i = k * 2
        step(i, xb0, pb0, xb1, pb1)
        step(i + 1, xb1, pb1, xb0, pb0)
        return 0

    lax.fori_loop(0, _NCHUNK // 2, loop, 0)
    wait_out(xb1)


def kernel(x, pos_table):
    B, T, D = x.shape
    xf = x.reshape(B * T, D)
    mesh = plsc.VectorSubcoreMesh(core_axis_name="c", subcore_axis_name="s")
    run = pl.kernel(
        _sc_body,
        out_type=jax.ShapeDtypeStruct((_ROWS, _D), jnp.float32),
        mesh=mesh,
        scratch_types=[
            pltpu.VMEM((_CHR, _D), jnp.float32),
            pltpu.VMEM((_CHR, _D), jnp.float32),
            pltpu.VMEM((_CHR, _D), jnp.float32),
            pltpu.VMEM((_CHR, _D), jnp.float32),
            pltpu.SemaphoreType.DMA,
            pltpu.SemaphoreType.DMA,
        ],
    )
    out = run(xf, pos_table[:T])
    return out.reshape(B, T, D)
